# Initial kernel scaffold; baseline (speedup 1.0000x reference)
import jax
import jax.numpy as jnp
from jax.experimental import pallas as pl


def kernel(inputs, offsets, W, b):
    raise NotImplementedError("write your pallas kernel here")


# speedup vs baseline: 14.5794x; 14.5794x over previous
import functools

import jax
import jax.numpy as jnp
from jax import lax
from jax.experimental import pallas as pl
from jax.experimental.pallas import tpu as pltpu

B, T, CIN, N = 8, 32, 132, 64
HID, OFF, TOPK = 256, 4, 16
BN = B * N
FAN = CIN + OFF + HID


def _step_kernel(xs_ref, offs_ref, past_ref, W_ref, b_ref,
                 out_ref, inds_ref, H, C, SEL):
    t = pl.program_id(0)

    @pl.when(t == 0)
    def _():
        H[...] = jnp.zeros_like(H)
        C[...] = jnp.zeros_like(C)

    x = xs_ref[0]
    pos = x[:OFF]
    cent = pos - offs_ref[0]
    past = past_ref[0]

    c3 = cent.reshape(OFF, B, N)
    p3 = past.reshape(OFF, B, N)
    d = jnp.zeros((B, N, N), dtype=jnp.float32)
    for c in range(OFF):
        diff = c3[c][:, :, None] - p3[c][:, None, :]
        d = d + diff * diff

    iota_m = lax.broadcasted_iota(jnp.int32, (B, N, N), 2)
    work = d
    for k in range(TOPK):
        mn = jnp.min(work, axis=2, keepdims=True)
        sel = jnp.min(jnp.where(work == mn, iota_m, N), axis=2)
        sel = sel.astype(jnp.int32)
        inds_ref[0, k] = sel
        SEL[:, k * N:(k + 1) * N] = sel
        work = jnp.where(iota_m == sel[:, :, None], jnp.inf, work)

    Wx = W_ref[:, :CIN]
    Wp = W_ref[:, CIN:CIN + OFF]
    Wh = W_ref[:, CIN:]
    A = (jnp.dot(Wx, x, preferred_element_type=jnp.float32)
         - jnp.dot(Wp, pos, preferred_element_type=jnp.float32)
         + b_ref[...])
    Hh = jnp.dot(Wh, H[...], preferred_element_type=jnp.float32)

    iota_g = lax.broadcasted_iota(jnp.int32, (N, TOPK * N), 0)
    for bb in range(B):
        cols = slice(bb * N, (bb + 1) * N)
        Gb = (iota_g == SEL[bb][None, :]).astype(jnp.float32)
        Zb = jnp.dot(Hh[:, cols], Gb, preferred_element_type=jnp.float32)
        Cgb = jnp.dot(C[:, cols], Gb, preferred_element_type=jnp.float32)
        At = jnp.concatenate([A[:, cols]] * TOPK, axis=1)
        z = Zb + At
        zi = z[0:HID]
        zf = z[HID:2 * HID]
        zo = z[2 * HID:3 * HID]
        zg = z[3 * HID:4 * HID]
        cn = (jax.nn.sigmoid(zf) * Cgb
              + jax.nn.sigmoid(zi) * jnp.tanh(zg))
        hn = jax.nn.sigmoid(zo) * jnp.tanh(cn)
        cmax = cn[:, 0:N]
        hmax = hn[:, 0:N]
        for k in range(1, TOPK):
            ks = slice(k * N, (k + 1) * N)
            cmax = jnp.maximum(cmax, cn[:, ks])
            hmax = jnp.maximum(hmax, hn[:, ks])
        C[:, cols] = cmax
        H[OFF:, cols] = hmax
        out_ref[0, OFF:, cols] = hmax

    H[:OFF, :] = pos
    out_ref[0, :OFF, :] = pos


@jax.jit
def kernel(inputs, offsets, W, b):
    xs = jnp.transpose(inputs, (1, 2, 0, 3)).reshape(T, CIN, BN)
    offs = jnp.transpose(offsets, (1, 2, 0, 3)).reshape(T, OFF, BN)
    pos = xs[:, :OFF]
    past = jnp.concatenate([pos[:1], pos[:-1]], axis=0)
    b2 = b.reshape(4 * HID, 1)

    outs, inds = pl.pallas_call(
        _step_kernel,
        grid=(T,),
        in_specs=[
            pl.BlockSpec((1, CIN, BN), lambda t: (t, 0, 0)),
            pl.BlockSpec((1, OFF, BN), lambda t: (t, 0, 0)),
            pl.BlockSpec((1, OFF, BN), lambda t: (t, 0, 0)),
            pl.BlockSpec((4 * HID, FAN), lambda t: (0, 0)),
            pl.BlockSpec((4 * HID, 1), lambda t: (0, 0)),
        ],
        out_specs=[
            pl.BlockSpec((1, OFF + HID, BN), lambda t: (t, 0, 0)),
            pl.BlockSpec((1, TOPK, B, N), lambda t: (t, 0, 0, 0)),
        ],
        out_shape=[
            jax.ShapeDtypeStruct((T, OFF + HID, BN), jnp.float32),
            jax.ShapeDtypeStruct((T, TOPK, B, N), jnp.int32),
        ],
        scratch_shapes=[
            pltpu.VMEM((OFF + HID, BN), jnp.float32),
            pltpu.VMEM((HID, BN), jnp.float32),
            pltpu.VMEM((B, TOPK * N), jnp.int32),
        ],
        compiler_params=pltpu.CompilerParams(
            dimension_semantics=("arbitrary",),
        ),
    )(xs, offs, past, W, b2)

    out = jnp.transpose(outs.reshape(T, OFF + HID, B, N), (2, 0, 1, 3))
    ind = jnp.transpose(inds, (2, 0, 3, 1))
    return out, ind


# speedup vs baseline: 19.0529x; 1.3068x over previous
import functools

import jax
import jax.numpy as jnp
from jax import lax
from jax.experimental import pallas as pl
from jax.experimental.pallas import tpu as pltpu
from jax.experimental.pallas import tpu_sc as plsc

B, T, CIN, N = 8, 32, 132, 64
HID, OFF, TOPK = 256, 4, 16
BN = B * N
KN = TOPK * N
FAN = CIN + OFF + HID
NPAIR = T * B
NWORK = 32
PPW = NPAIR // NWORK


def _knn_sc_body(cent_hbm, past_hbm, sel_hbm, cbuf, pbuf, selbuf):
    wid = lax.axis_index("s") * 2 + lax.axis_index("c")
    iotav = lax.iota(jnp.int32, 16)

    def merge(ak, av, bk, bv):
        rbk = lax.rev(bk, (0,))
        rbv = lax.rev(bv, (0,))
        m = ak <= rbk
        lk = jnp.where(m, ak, rbk)
        lv = jnp.where(m, av, rbv)
        return plsc.sort_key_val(lk, lv)

    def pair_body(i, carry):
        pair = wid * PPW + i
        pltpu.sync_copy(cent_hbm.at[pair], cbuf)
        pltpu.sync_copy(past_hbm.at[pair], pbuf)
        pvt = [[pbuf[pl.ds(c * N + j * 16, 16)] for j in range(4)]
               for c in range(OFF)]

        def g_body(g, gcarry):
            cvecs = [cbuf[pl.ds(c * N + g * 16, 16)] for c in range(OFF)]
            for q in range(16):
                runs = []
                for j in range(4):
                    acc = None
                    for c in range(OFF):
                        diff = cvecs[c][q] - pvt[c][j]
                        sq = diff * diff
                        acc = sq if acc is None else acc + sq
                    runs.append(plsc.sort_key_val(acc, iotav + j * 16))
                k0, v0 = merge(*runs[0], *runs[1])
                k1, v1 = merge(*runs[2], *runs[3])
                _, fv = merge(k0, v0, k1, v1)
                plsc.store_scatter(selbuf, [iotav * N + (g * 16 + q)], fv)
            return gcarry

        lax.fori_loop(0, 4, g_body, 0)
        pltpu.sync_copy(selbuf, sel_hbm.at[pair])
        return carry

    lax.fori_loop(0, PPW, pair_body, 0)


@functools.cache
def _make_knn_sc():
    return functools.partial(
        pl.kernel,
        out_type=jax.ShapeDtypeStruct((NPAIR, KN), jnp.int32),
        mesh=plsc.VectorSubcoreMesh(
            core_axis_name="c", subcore_axis_name="s", num_cores=2),
        scratch_types=[
            pltpu.VMEM((OFF * N,), jnp.float32),
            pltpu.VMEM((OFF * N,), jnp.float32),
            pltpu.VMEM((KN,), jnp.int32),
        ],
        compiler_params=pltpu.CompilerParams(needs_layout_passes=False),
    )(_knn_sc_body)


def _step_kernel(xs_ref, sel_ref, W_ref, b_ref, out_ref, H, C):
    t = pl.program_id(0)

    @pl.when(t == 0)
    def _():
        H[...] = jnp.zeros_like(H)
        C[...] = jnp.zeros_like(C)

    x = xs_ref[0]
    pos = x[:OFF]

    Wx = W_ref[:, :CIN]
    Wp = W_ref[:, CIN:CIN + OFF]
    Wh = W_ref[:, CIN:]
    A = (jnp.dot(Wx, x, preferred_element_type=jnp.float32)
         - jnp.dot(Wp, pos, preferred_element_type=jnp.float32)
         + b_ref[...])
    Hh = jnp.dot(Wh, H[...], preferred_element_type=jnp.float32)

    iota_g = lax.broadcasted_iota(jnp.int32, (N, KN), 0)
    E = (iota_g == lax.broadcasted_iota(jnp.int32, (N, KN), 1) % N
         ).astype(jnp.float32)
    for bb in range(B):
        cols = slice(bb * N, (bb + 1) * N)
        Gb = (iota_g == sel_ref[0, bb][None, :]).astype(jnp.float32)
        lhs = jnp.concatenate([Hh[:, cols], A[:, cols]], axis=1)
        rhs = jnp.concatenate([Gb, E], axis=0)
        z = jnp.dot(lhs, rhs, preferred_element_type=jnp.float32)
        Cgb = jnp.dot(C[:, cols], Gb, preferred_element_type=jnp.float32)
        zi = z[0:HID]
        zf = z[HID:2 * HID]
        zo = z[2 * HID:3 * HID]
        zg = z[3 * HID:4 * HID]
        cn = (jax.nn.sigmoid(zf) * Cgb
              + jax.nn.sigmoid(zi) * jnp.tanh(zg))
        hn = jax.nn.sigmoid(zo) * jnp.tanh(cn)
        cmax = cn[:, 0:N]
        hmax = hn[:, 0:N]
        for k in range(1, TOPK):
            ks = slice(k * N, (k + 1) * N)
            cmax = jnp.maximum(cmax, cn[:, ks])
            hmax = jnp.maximum(hmax, hn[:, ks])
        C[:, cols] = cmax
        H[OFF:, cols] = hmax
        out_ref[0, OFF:, cols] = hmax

    H[:OFF, :] = pos
    out_ref[0, :OFF, :] = pos


@jax.jit
def kernel(inputs, offsets, W, b):
    xs = jnp.transpose(inputs, (1, 2, 0, 3)).reshape(T, CIN, BN)
    pos = xs[:, :OFF]

    cent = (pos - jnp.transpose(offsets, (1, 2, 0, 3)).reshape(T, OFF, BN))
    past = jnp.concatenate([pos[:1], pos[:-1]], axis=0)
    cent_p = jnp.transpose(cent.reshape(T, OFF, B, N), (0, 2, 1, 3))
    past_p = jnp.transpose(past.reshape(T, OFF, B, N), (0, 2, 1, 3))
    sel = _make_knn_sc()(cent_p.reshape(NPAIR, OFF * N),
                         past_p.reshape(NPAIR, OFF * N))

    b2 = b.reshape(4 * HID, 1)
    sel3 = sel.reshape(T, B, KN)

    outs = pl.pallas_call(
        _step_kernel,
        grid=(T,),
        in_specs=[
            pl.BlockSpec((1, CIN, BN), lambda t: (t, 0, 0)),
            pl.BlockSpec((1, B, KN), lambda t: (t, 0, 0)),
            pl.BlockSpec((4 * HID, FAN), lambda t: (0, 0)),
            pl.BlockSpec((4 * HID, 1), lambda t: (0, 0)),
        ],
        out_specs=pl.BlockSpec((1, OFF + HID, BN), lambda t: (t, 0, 0)),
        out_shape=jax.ShapeDtypeStruct((T, OFF + HID, BN), jnp.float32),
        scratch_shapes=[
            pltpu.VMEM((OFF + HID, BN), jnp.float32),
            pltpu.VMEM((HID, BN), jnp.float32),
        ],
        compiler_params=pltpu.CompilerParams(
            dimension_semantics=("arbitrary",),
        ),
    )(xs, sel3, W, b2)

    out = jnp.transpose(outs.reshape(T, OFF + HID, B, N), (2, 0, 1, 3))
    ind = jnp.transpose(sel.reshape(T, B, TOPK, N), (1, 0, 3, 2))
    return out, ind


# speedup vs baseline: 19.5905x; 1.0282x over previous
import functools

import jax
import jax.numpy as jnp
from jax import lax
from jax.experimental import pallas as pl
from jax.experimental.pallas import tpu as pltpu
from jax.experimental.pallas import tpu_sc as plsc

B, T, CIN, N = 8, 32, 132, 64
HID, OFF, TOPK = 256, 4, 16
BN = B * N
KN = TOPK * N
FAN = CIN + OFF + HID
NPAIR = T * B
NWORK = 32
PPW = NPAIR // NWORK


def _knn_sc_body(cent_hbm, past_hbm, sel_hbm, cbuf, pbuf, selbuf):
    wid = lax.axis_index("s") * 2 + lax.axis_index("c")
    iotav = lax.iota(jnp.int32, 16)

    def merge(ak, av, bk, bv):
        rbk = lax.rev(bk, (0,))
        rbv = lax.rev(bv, (0,))
        m = ak <= rbk
        lk = jnp.where(m, ak, rbk)
        lv = jnp.where(m, av, rbv)
        return plsc.sort_key_val(lk, lv)

    def pair_body(i, carry):
        pair = wid * PPW + i
        pltpu.sync_copy(cent_hbm.at[pair], cbuf)
        pltpu.sync_copy(past_hbm.at[pair], pbuf)
        pvt = [[pbuf[pl.ds(c * N + j * 16, 16)] for j in range(4)]
               for c in range(OFF)]

        def g_body(g, gcarry):
            cvecs = [cbuf[pl.ds(c * N + g * 16, 16)] for c in range(OFF)]
            for q in range(16):
                runs = []
                for j in range(4):
                    acc = None
                    for c in range(OFF):
                        diff = cvecs[c][q] - pvt[c][j]
                        sq = diff * diff
                        acc = sq if acc is None else acc + sq
                    runs.append(plsc.sort_key_val(acc, iotav + j * 16))
                k0, v0 = merge(*runs[0], *runs[1])
                k1, v1 = merge(*runs[2], *runs[3])
                _, fv = merge(k0, v0, k1, v1)
                plsc.store_scatter(selbuf, [iotav * N + (g * 16 + q)], fv)
            return gcarry

        lax.fori_loop(0, 4, g_body, 0)
        pltpu.sync_copy(selbuf, sel_hbm.at[pair])
        return carry

    lax.fori_loop(0, PPW, pair_body, 0)


@functools.cache
def _make_knn_sc():
    return functools.partial(
        pl.kernel,
        out_type=jax.ShapeDtypeStruct((NPAIR, KN), jnp.int32),
        mesh=plsc.VectorSubcoreMesh(
            core_axis_name="c", subcore_axis_name="s", num_cores=2),
        scratch_types=[
            pltpu.VMEM((OFF * N,), jnp.float32),
            pltpu.VMEM((OFF * N,), jnp.float32),
            pltpu.VMEM((KN,), jnp.int32),
        ],
        compiler_params=pltpu.CompilerParams(needs_layout_passes=False),
    )(_knn_sc_body)


def _step_kernel(xs_ref, sel_ref, W_ref, b_ref, out_ref, H, C):
    t = pl.program_id(0)

    @pl.when(t == 0)
    def _():
        H[...] = jnp.zeros_like(H)
        C[...] = jnp.zeros_like(C)

    x = xs_ref[0]
    pos = x[:OFF]

    Wx = W_ref[:, :CIN]
    Wp = W_ref[:, CIN:CIN + OFF]
    Wh = W_ref[:, CIN:]
    A = (jnp.dot(Wx, x, preferred_element_type=jnp.float32)
         - jnp.dot(Wp, pos, preferred_element_type=jnp.float32)
         + b_ref[...])
    Hh = jnp.dot(Wh, H[...], preferred_element_type=jnp.float32)

    iota_g = lax.broadcasted_iota(jnp.int32, (N, KN), 0)
    E = (iota_g == lax.broadcasted_iota(jnp.int32, (N, KN), 1) % N
         ).astype(jnp.float32)
    zeroN = jnp.zeros((HID, N), dtype=jnp.float32)
    for bb in range(B):
        cols = slice(bb * N, (bb + 1) * N)
        Gb = (iota_g == sel_ref[0, bb][None, :]).astype(jnp.float32)
        lhs = jnp.concatenate(
            [jnp.concatenate([Hh[:, cols], A[:, cols]], axis=1),
             jnp.concatenate([C[:, cols], zeroN], axis=1)], axis=0)
        rhs = jnp.concatenate([Gb, E], axis=0)
        zz = jnp.dot(lhs, rhs, preferred_element_type=jnp.float32)
        zi = zz[0:HID]
        zf = zz[HID:2 * HID]
        zo = zz[2 * HID:3 * HID]
        zg = zz[3 * HID:4 * HID]
        Cgb = zz[4 * HID:]
        cn = (jax.nn.sigmoid(zf) * Cgb
              + jax.nn.sigmoid(zi) * jnp.tanh(zg))
        hn = jax.nn.sigmoid(zo) * jnp.tanh(cn)
        w = KN
        while w > N:
            w //= 2
            cn = jnp.maximum(cn[:, :w], cn[:, w:2 * w])
            hn = jnp.maximum(hn[:, :w], hn[:, w:2 * w])
        C[:, cols] = cn
        H[OFF:, cols] = hn
        out_ref[0, OFF:, cols] = hn

    H[:OFF, :] = pos
    out_ref[0, :OFF, :] = pos


@jax.jit
def kernel(inputs, offsets, W, b):
    xs = jnp.transpose(inputs, (1, 2, 0, 3)).reshape(T, CIN, BN)
    pos = xs[:, :OFF]

    cent = (pos - jnp.transpose(offsets, (1, 2, 0, 3)).reshape(T, OFF, BN))
    past = jnp.concatenate([pos[:1], pos[:-1]], axis=0)
    cent_p = jnp.transpose(cent.reshape(T, OFF, B, N), (0, 2, 1, 3))
    past_p = jnp.transpose(past.reshape(T, OFF, B, N), (0, 2, 1, 3))
    sel = _make_knn_sc()(cent_p.reshape(NPAIR, OFF * N),
                         past_p.reshape(NPAIR, OFF * N))

    b2 = b.reshape(4 * HID, 1)
    sel3 = sel.reshape(T, B, KN)

    outs = pl.pallas_call(
        _step_kernel,
        grid=(T,),
        in_specs=[
            pl.BlockSpec((1, CIN, BN), lambda t: (t, 0, 0)),
            pl.BlockSpec((1, B, KN), lambda t: (t, 0, 0)),
            pl.BlockSpec((4 * HID, FAN), lambda t: (0, 0)),
            pl.BlockSpec((4 * HID, 1), lambda t: (0, 0)),
        ],
        out_specs=pl.BlockSpec((1, OFF + HID, BN), lambda t: (t, 0, 0)),
        out_shape=jax.ShapeDtypeStruct((T, OFF + HID, BN), jnp.float32),
        scratch_shapes=[
            pltpu.VMEM((OFF + HID, BN), jnp.float32),
            pltpu.VMEM((HID, BN), jnp.float32),
        ],
        compiler_params=pltpu.CompilerParams(
            dimension_semantics=("arbitrary",),
        ),
    )(xs, sel3, W, b2)

    out = jnp.transpose(outs.reshape(T, OFF + HID, B, N), (2, 0, 1, 3))
    ind = jnp.transpose(sel.reshape(T, B, TOPK, N), (1, 0, 3, 2))
    return out, ind
